# baseline (device time: 12913 ns/iter reference)
import jax
import jax.numpy as jnp
from jax import lax
from jax.experimental import pallas as pl
from jax.experimental.pallas import tpu as pltpu

N_DEV = 4
B = 2
SQ = 128
DH = 64
H_LOC = 4
D_LOC = H_LOC * DH
D_MODEL = 512


def kernel(x, Wq, K_ext, V_ext, Wo):
    K_t = jnp.transpose(K_ext, (0, 2, 3, 1))
    V_t = jnp.transpose(V_ext, (0, 2, 3, 1))

    hbm = lambda a: pltpu.with_memory_space_constraint(a, pltpu.MemorySpace.HBM)
    x, Wq, K_t, V_t, Wo = hbm(x), hbm(Wq), hbm(K_t), hbm(V_t), hbm(Wo)

    out_buf = hbm(jnp.zeros((B, SQ, D_MODEL), jnp.float32))

    def body(x_hbm, wq_hbm, k_hbm, v_hbm, wo_hbm, outbuf_hbm, out_ref,
             xv, wqv, kv, vv, wov, outv, comm_ref,
             dma_sems, send_sems, recv_sems):
        del outbuf_hbm
        my_pos = lax.axis_index("i")
        left = lax.rem(my_pos + N_DEV - 1, N_DEV)
        right = lax.rem(my_pos + 1, N_DEV)
        opp = lax.rem(my_pos + 2, N_DEV)
        col0 = my_pos * D_LOC

        barrier_sem = pltpu.get_barrier_semaphore()
        for nbr in (left, right, opp):
            pl.semaphore_signal(
                barrier_sem, inc=1,
                device_id=(nbr,), device_id_type=pl.DeviceIdType.MESH,
            )

        cp_x = pltpu.make_async_copy(x_hbm, xv, dma_sems.at[0])
        cp_x.start()
        cp_wq = pltpu.make_async_copy(
            wq_hbm.at[:, pl.ds(col0, D_LOC)], wqv, dma_sems.at[1])
        cp_wq.start()
        cp_k = pltpu.make_async_copy(k_hbm, kv, dma_sems.at[2])
        cp_k.start()
        cp_v = pltpu.make_async_copy(v_hbm, vv, dma_sems.at[3])
        cp_v.start()
        cp_wo = []
        for t, org in enumerate((my_pos, left, right, opp)):
            c = pltpu.make_async_copy(
                wo_hbm.at[pl.ds(org * D_LOC, D_LOC), :], wov.at[t],
                dma_sems.at[4 + t])
            c.start()
            cp_wo.append(c)

        cp_x.wait()
        cp_wq.wait()
        wq_bf = wqv[:].astype(jnp.bfloat16)
        q_alls = []
        for b in range(B):
            xb = xv[b].astype(jnp.bfloat16)
            qa = jnp.dot(xb, wq_bf,
                         preferred_element_type=jnp.float32)
            q_alls.append(qa.astype(jnp.bfloat16))
        cp_k.wait()
        cp_v.wait()

        def attn_batch(b):
            q_all = q_alls[b]
            ctxs = []
            for j in range(H_LOC):
                q = q_all[:, j * DH:(j + 1) * DH]
                kT = kv[b, j].astype(jnp.bfloat16)
                vT = vv[b, j].astype(jnp.bfloat16)
                s = lax.dot_general(
                    q, kT, (((1,), (0,)), ((), ())),
                    preferred_element_type=jnp.float32) * 0.125
                e = jnp.exp(s)
                r = 1.0 / jnp.sum(e, axis=-1, keepdims=True)
                cj = lax.dot_general(
                    e.astype(jnp.bfloat16), vT, (((1,), (1,)), ((), ())),
                    preferred_element_type=jnp.float32)
                ctxs.append(cj * r)
            return jnp.concatenate(ctxs, axis=1).astype(jnp.bfloat16)

        def pushes(b):
            out = []
            for i, (tgt, slot) in enumerate(((opp, 3), (right, 1), (left, 2))):
                r = pltpu.make_async_remote_copy(
                    src_ref=comm_ref.at[0, b], dst_ref=comm_ref.at[slot, b],
                    send_sem=send_sems.at[3 * b + i],
                    recv_sem=recv_sems.at[3 * b + i],
                    device_id=(tgt,), device_id_type=pl.DeviceIdType.MESH,
                )
                r.start()
                out.append(r)
            return out

        ctx0 = attn_batch(0)
        comm_ref[0, 0] = ctx0
        pl.semaphore_wait(barrier_sem, 3)
        p0 = pushes(0)
        ctx1 = attn_batch(1)
        comm_ref[0, 1] = ctx1
        p1 = pushes(1)

        cp_wo[0].wait()
        wo_own = wov[0].astype(jnp.bfloat16)
        acc = [jnp.dot(c, wo_own, preferred_element_type=jnp.float32)
               for c in (ctx0, ctx1)]

        cp_wo[1].wait()
        cp_wo[2].wait()
        wo_left = wov[1].astype(jnp.bfloat16)
        wo_right = wov[2].astype(jnp.bfloat16)
        p0[2].wait_recv()
        p0[1].wait_recv()
        acc[0] = acc[0] + jnp.dot(
            comm_ref[1, 0], wo_left, preferred_element_type=jnp.float32)
        acc[0] = acc[0] + jnp.dot(
            comm_ref[2, 0], wo_right, preferred_element_type=jnp.float32)
        p1[2].wait_recv()
        p1[1].wait_recv()
        acc[1] = acc[1] + jnp.dot(
            comm_ref[1, 1], wo_left, preferred_element_type=jnp.float32)
        acc[1] = acc[1] + jnp.dot(
            comm_ref[2, 1], wo_right, preferred_element_type=jnp.float32)

        cp_wo[3].wait()
        wo_opp = wov[3].astype(jnp.bfloat16)
        p0[0].wait_recv()
        acc[0] = acc[0] + jnp.dot(
            comm_ref[3, 0], wo_opp, preferred_element_type=jnp.float32)
        outv[0] = acc[0]
        cp_out0 = pltpu.make_async_copy(
            outv.at[0], out_ref.at[0], dma_sems.at[8])
        cp_out0.start()
        p1[0].wait_recv()
        acc[1] = acc[1] + jnp.dot(
            comm_ref[3, 1], wo_opp, preferred_element_type=jnp.float32)
        outv[1] = acc[1]
        cp_out1 = pltpu.make_async_copy(
            outv.at[1], out_ref.at[1], dma_sems.at[9])
        cp_out1.start()
        cp_out0.wait()
        cp_out1.wait()

        for r in p0 + p1:
            r.wait_send()

    return pl.pallas_call(
        body,
        out_shape=jax.ShapeDtypeStruct((B, SQ, D_MODEL), jnp.float32),
        in_specs=[pl.BlockSpec(memory_space=pl.ANY)] * 6,
        out_specs=pl.BlockSpec(memory_space=pl.ANY),
        input_output_aliases={5: 0},
        scratch_shapes=[
            pltpu.VMEM((B, SQ, D_MODEL), jnp.float32),
            pltpu.VMEM((D_MODEL, D_LOC), jnp.float32),
            pltpu.VMEM((B, H_LOC, DH, SQ), jnp.float32),
            pltpu.VMEM((B, H_LOC, DH, SQ), jnp.float32),
            pltpu.VMEM((4, D_LOC, D_MODEL), jnp.float32),
            pltpu.VMEM((B, SQ, D_MODEL), jnp.float32),
            pltpu.VMEM((4, B, SQ, D_LOC), jnp.bfloat16),
            pltpu.SemaphoreType.DMA((10,)),
            pltpu.SemaphoreType.DMA((6,)),
            pltpu.SemaphoreType.DMA((6,)),
        ],
        compiler_params=pltpu.CompilerParams(collective_id=0),
    )(x, Wq, K_t, V_t, Wo, out_buf)


# device time: 10730 ns/iter; 1.2034x vs baseline; 1.2034x over previous
import jax
import jax.numpy as jnp
from jax import lax
from jax.experimental import pallas as pl
from jax.experimental.pallas import tpu as pltpu

N_DEV = 4
B = 2
SQ = 128
DH = 64
H_LOC = 4
D_LOC = H_LOC * DH
D_MODEL = 512


def kernel(x, Wq, K_ext, V_ext, Wo):
    K_t = jnp.transpose(K_ext, (0, 2, 3, 1))
    V_t = jnp.transpose(V_ext, (0, 2, 3, 1))

    hbm = lambda a: pltpu.with_memory_space_constraint(a, pltpu.MemorySpace.HBM)
    x, Wq, K_t, V_t, Wo = hbm(x), hbm(Wq), hbm(K_t), hbm(V_t), hbm(Wo)

    def body(x_hbm, wq_hbm, k_hbm, v_hbm, wo_hbm, out_ref,
             xv, wqv, kv, vv, wov, comm_ref,
             dma_sems, send_sems, recv_sems):
        my_pos = lax.axis_index("i")
        left = lax.rem(my_pos + N_DEV - 1, N_DEV)
        right = lax.rem(my_pos + 1, N_DEV)
        opp = lax.rem(my_pos + 2, N_DEV)
        col0 = my_pos * D_LOC

        barrier_sem = pltpu.get_barrier_semaphore()
        for nbr in (left, right, opp):
            pl.semaphore_signal(
                barrier_sem, inc=1,
                device_id=(nbr,), device_id_type=pl.DeviceIdType.MESH,
            )

        cp_x, cp_k, cp_v = [], [], []
        for b in range(B):
            c = pltpu.make_async_copy(
                x_hbm.at[b], xv.at[b], dma_sems.at[3 * b])
            c.start()
            cp_x.append(c)
            c = pltpu.make_async_copy(
                k_hbm.at[b], kv.at[b], dma_sems.at[3 * b + 1])
            c.start()
            cp_k.append(c)
            c = pltpu.make_async_copy(
                v_hbm.at[b], vv.at[b], dma_sems.at[3 * b + 2])
            c.start()
            cp_v.append(c)
        cp_wq = pltpu.make_async_copy(
            wq_hbm.at[:, pl.ds(col0, D_LOC)], wqv, dma_sems.at[6])
        cp_wq.start()
        cp_wo = []
        for t, org in enumerate((my_pos, left, right, opp)):
            c = pltpu.make_async_copy(
                wo_hbm.at[pl.ds(org * D_LOC, D_LOC), :], wov.at[t],
                dma_sems.at[7 + t])
            c.start()
            cp_wo.append(c)

        cp_wq.wait()
        wq_bf = wqv[:].astype(jnp.bfloat16)

        def attn_batch(b):
            cp_x[b].wait()
            xb = xv[b].astype(jnp.bfloat16)
            q_all = jnp.dot(xb, wq_bf,
                            preferred_element_type=jnp.float32)
            q_all = q_all.astype(jnp.bfloat16)
            cp_k[b].wait()
            cp_v[b].wait()
            ctxs = []
            for j in range(H_LOC):
                q = q_all[:, j * DH:(j + 1) * DH]
                kT = kv[b, j].astype(jnp.bfloat16)
                vT = vv[b, j].astype(jnp.bfloat16)
                s = lax.dot_general(
                    q, kT, (((1,), (0,)), ((), ())),
                    preferred_element_type=jnp.float32) * 0.125
                e = jnp.exp(s)
                r = 1.0 / jnp.sum(e, axis=-1, keepdims=True)
                cj = lax.dot_general(
                    e.astype(jnp.bfloat16), vT, (((1,), (1,)), ((), ())),
                    preferred_element_type=jnp.float32)
                ctxs.append(cj * r)
            return jnp.concatenate(ctxs, axis=1).astype(jnp.bfloat16)

        def pushes(b):
            out = []
            for i, (tgt, slot) in enumerate(((opp, 3), (right, 1), (left, 2))):
                r = pltpu.make_async_remote_copy(
                    src_ref=comm_ref.at[0, b], dst_ref=comm_ref.at[slot, b],
                    send_sem=send_sems.at[3 * b + i],
                    recv_sem=recv_sems.at[3 * b + i],
                    device_id=(tgt,), device_id_type=pl.DeviceIdType.MESH,
                )
                r.start()
                out.append(r)
            return out

        ctx0 = attn_batch(0)
        comm_ref[0, 0] = ctx0
        pl.semaphore_wait(barrier_sem, 3)
        p0 = pushes(0)
        ctx1 = attn_batch(1)
        comm_ref[0, 1] = ctx1
        p1 = pushes(1)

        cp_wo[0].wait()
        wo_own = wov[0].astype(jnp.bfloat16)
        acc = [jnp.dot(c, wo_own, preferred_element_type=jnp.float32)
               for c in (ctx0, ctx1)]

        cp_wo[1].wait()
        cp_wo[2].wait()
        wo_left = wov[1].astype(jnp.bfloat16)
        wo_right = wov[2].astype(jnp.bfloat16)
        p0[2].wait_recv()
        p0[1].wait_recv()
        acc[0] = acc[0] + jnp.dot(
            comm_ref[1, 0], wo_left, preferred_element_type=jnp.float32)
        acc[0] = acc[0] + jnp.dot(
            comm_ref[2, 0], wo_right, preferred_element_type=jnp.float32)
        p1[2].wait_recv()
        p1[1].wait_recv()
        acc[1] = acc[1] + jnp.dot(
            comm_ref[1, 1], wo_left, preferred_element_type=jnp.float32)
        acc[1] = acc[1] + jnp.dot(
            comm_ref[2, 1], wo_right, preferred_element_type=jnp.float32)

        cp_wo[3].wait()
        wo_opp = wov[3].astype(jnp.bfloat16)
        p0[0].wait_recv()
        acc[0] = acc[0] + jnp.dot(
            comm_ref[3, 0], wo_opp, preferred_element_type=jnp.float32)
        out_ref[0] = acc[0]
        p1[0].wait_recv()
        acc[1] = acc[1] + jnp.dot(
            comm_ref[3, 1], wo_opp, preferred_element_type=jnp.float32)
        out_ref[1] = acc[1]

        for r in p0 + p1:
            r.wait_send()

    return pl.pallas_call(
        body,
        out_shape=jax.ShapeDtypeStruct((B, SQ, D_MODEL), jnp.float32),
        in_specs=[pl.BlockSpec(memory_space=pl.ANY)] * 5,
        out_specs=pl.BlockSpec(memory_space=pltpu.VMEM),
        scratch_shapes=[
            pltpu.VMEM((B, SQ, D_MODEL), jnp.float32),
            pltpu.VMEM((D_MODEL, D_LOC), jnp.float32),
            pltpu.VMEM((B, H_LOC, DH, SQ), jnp.float32),
            pltpu.VMEM((B, H_LOC, DH, SQ), jnp.float32),
            pltpu.VMEM((4, D_LOC, D_MODEL), jnp.float32),
            pltpu.VMEM((4, B, SQ, D_LOC), jnp.bfloat16),
            pltpu.SemaphoreType.DMA((11,)),
            pltpu.SemaphoreType.DMA((6,)),
            pltpu.SemaphoreType.DMA((6,)),
        ],
        compiler_params=pltpu.CompilerParams(collective_id=0),
    )(x, Wq, K_t, V_t, Wo)
